# Initial kernel scaffold; baseline (speedup 1.0000x reference)
import jax
import jax.numpy as jnp
from jax.experimental import pallas as pl


def kernel(x, idx_mask):
    raise NotImplementedError("write your pallas kernel here")


# speedup vs baseline: 1.2941x; 1.2941x over previous
import functools

import jax
import jax.numpy as jnp
from jax import lax
from jax.experimental import pallas as pl
from jax.experimental.pallas import tpu as pltpu
from jax.experimental.pallas import tpu_sc as plsc

_N = 100000
_D = 128
_M = 10000


_BLK = 2000

def _copy_sum_body(x_ref, out_ref, tok_ref, acc_ref):
    i = pl.program_id(0)
    blk = x_ref[...]
    out_ref[...] = blk

    @pl.when(i == 0)
    def _():
        acc_ref[...] = jnp.zeros_like(acc_ref)

    acc_ref[...] += jnp.sum(blk.reshape(_BLK // 8, 8, _D), axis=0)

    @pl.when(i == pl.num_programs(0) - 1)
    def _():
        tok_ref[...] = jnp.sum(acc_ref[...], axis=0, keepdims=True) * (1.0 / _N)


def _copy_and_mean(x):
    return pl.pallas_call(
        _copy_sum_body,
        grid=(_N // _BLK,),
        in_specs=[pl.BlockSpec((_BLK, _D), lambda i: (i, 0))],
        out_specs=[
            pl.BlockSpec((_BLK, _D), lambda i: (i, 0)),
            pl.BlockSpec((1, _D), lambda i: (0, 0)),
        ],
        out_shape=[
            jax.ShapeDtypeStruct((_N, _D), jnp.float32),
            jax.ShapeDtypeStruct((1, _D), jnp.float32),
        ],
        scratch_shapes=[pltpu.VMEM((8, _D), jnp.float32)],
    )(x)


_NC = 2
_NS = 16
_NW = _NC * _NS
_CH = 80
_NCH = 4
_PER = _CH * _NCH
_PAD = _NW * _PER

def _sc_scatter_body(out_hbm, tok_hbm, idx_hbm, i0, i1, i2, i3, tok_v, rows_v, sem):
    w = lax.axis_index("s") * _NC + lax.axis_index("c")
    idx_refs = (i0, i1, i2, i3)
    for j, iv in enumerate(idx_refs):
        pltpu.sync_copy(idx_hbm.at[w, j], iv)
    pltpu.sync_copy(tok_hbm, tok_v)
    tvec = [tok_v[pl.ds(16 * j, 16)] for j in range(_D // 16)]

    def _fill(i, _):
        for j in range(_D // 16):
            rows_v[i, pl.ds(16 * j, 16)] = tvec[j]
        return 0

    lax.fori_loop(0, _CH, _fill, 0)
    copies = [
        pltpu.async_copy(rows_v, out_hbm.at[iv], sem) for iv in idx_refs
    ]
    for c in copies:
        c.wait()


@functools.cache
def _get_sc_scatter():
    mesh = plsc.VectorSubcoreMesh(
        core_axis_name="c", subcore_axis_name="s",
        num_cores=_NC, num_subcores=_NS,
    )
    return pl.kernel(
        _sc_scatter_body,
        out_type=(),
        mesh=mesh,
        scratch_types=[
            *([pltpu.VMEM((_CH,), jnp.int32)] * _NCH),
            pltpu.VMEM((_D,), jnp.float32),
            pltpu.VMEM((_CH, _D), jnp.float32),
            pltpu.SemaphoreType.DMA,
        ],
    )


def kernel(x, idx_mask):
    out, tok = _copy_and_mean(x)
    idx_pad = jnp.concatenate(
        [idx_mask, jnp.broadcast_to(idx_mask[:1], (_PAD - _M,))]
    ).reshape(_NW, _NCH, _CH)
    out_ref = jax.new_ref(out)
    _get_sc_scatter()(out_ref, tok.reshape(_D), idx_pad)
    return out_ref[...]


# speedup vs baseline: 1.5942x; 1.2319x over previous
import functools

import jax
import jax.numpy as jnp
from jax import lax
from jax.experimental import pallas as pl
from jax.experimental.pallas import tpu as pltpu
from jax.experimental.pallas import tpu_sc as plsc

_N = 100000
_D = 128
_M = 10000


_BLK = 4000

def _copy_sum_body(x_ref, out_ref, tok_ref, acc_ref):
    i = pl.program_id(0)
    blk = x_ref[...]
    out_ref[...] = blk

    @pl.when(i == 0)
    def _():
        acc_ref[...] = jnp.zeros_like(acc_ref)

    acc_ref[...] += jnp.sum(blk.reshape(_BLK // 8, 8, _D), axis=0)

    @pl.when(i == pl.num_programs(0) - 1)
    def _():
        tok_ref[...] = jnp.sum(acc_ref[...], axis=0, keepdims=True) * (1.0 / _N)


def _copy_and_mean(x):
    return pl.pallas_call(
        _copy_sum_body,
        grid=(_N // _BLK,),
        in_specs=[pl.BlockSpec((_BLK, _D), lambda i: (i, 0))],
        out_specs=[
            pl.BlockSpec((_BLK, _D), lambda i: (i, 0)),
            pl.BlockSpec((1, _D), lambda i: (0, 0)),
        ],
        out_shape=[
            jax.ShapeDtypeStruct((_N, _D), jnp.float32),
            jax.ShapeDtypeStruct((1, _D), jnp.float32),
        ],
        scratch_shapes=[pltpu.VMEM((8, _D), jnp.float32)],
    )(x)


_NC = 2
_NS = 16
_NW = _NC * _NS
_CH = 80
_NCH = 4
_PER = _CH * _NCH
_PAD = _NW * _PER

def _sc_scatter_body(
    out_hbm, tok_hbm, idx_hbm, i0, i1, i2, i3, tok_v, rows_v, isem, sem
):
    w = lax.axis_index("s") * _NC + lax.axis_index("c")
    idx_refs = (i0, i1, i2, i3)
    idx_copies = [
        pltpu.async_copy(idx_hbm.at[w, j], iv, isem)
        for j, iv in enumerate(idx_refs)
    ]
    pltpu.sync_copy(tok_hbm, tok_v)
    tvec = [tok_v[pl.ds(16 * j, 16)] for j in range(_D // 16)]

    def _fill(i, _):
        for j in range(_D // 16):
            rows_v[i, pl.ds(16 * j, 16)] = tvec[j]
        return 0

    lax.fori_loop(0, _CH, _fill, 0)
    for c in idx_copies:
        c.wait()
    copies = [
        pltpu.async_copy(rows_v, out_hbm.at[iv], sem) for iv in idx_refs
    ]
    for c in copies:
        c.wait()


@functools.cache
def _get_sc_scatter():
    mesh = plsc.VectorSubcoreMesh(
        core_axis_name="c", subcore_axis_name="s",
        num_cores=_NC, num_subcores=_NS,
    )
    return pl.kernel(
        _sc_scatter_body,
        out_type=(),
        mesh=mesh,
        scratch_types=[
            *([pltpu.VMEM((_CH,), jnp.int32)] * _NCH),
            pltpu.VMEM((_D,), jnp.float32),
            pltpu.VMEM((_CH, _D), jnp.float32),
            pltpu.SemaphoreType.DMA,
            pltpu.SemaphoreType.DMA,
        ],
    )


def kernel(x, idx_mask):
    out, tok = _copy_and_mean(x)
    idx_pad = jnp.concatenate(
        [idx_mask, jnp.broadcast_to(idx_mask[:1], (_PAD - _M,))]
    ).reshape(_NW, _NCH, _CH)
    out_ref = jax.new_ref(out)
    _get_sc_scatter()(out_ref, tok.reshape(_D), idx_pad)
    return out_ref[...]
